# Initial kernel scaffold; baseline (speedup 1.0000x reference)
import jax
import jax.numpy as jnp
from jax.experimental import pallas as pl


def kernel(cosine, label):
    raise NotImplementedError("write your pallas kernel here")


# speedup vs baseline: 1.0618x; 1.0618x over previous
import jax
import jax.numpy as jnp
from jax.experimental import pallas as pl

_SCALE = 32.0
_MARGIN = 0.2


def _body(lab_ref, cos_ref, out_ref):
    cos = cos_ref[...]
    lab = lab_ref[...]
    cols = jax.lax.broadcasted_iota(jnp.int32, cos.shape, 1)
    mask = cols == lab
    out_ref[...] = jnp.where(mask, (cos - _MARGIN) * _SCALE, cos * _SCALE)


def kernel(cosine, label):
    B, C = cosine.shape
    RB = 16
    lab2 = label.astype(jnp.int32).reshape(B, 1)
    return pl.pallas_call(
        _body,
        grid=(B // RB,),
        in_specs=[
            pl.BlockSpec((RB, 1), lambda i: (i, 0)),
            pl.BlockSpec((RB, C), lambda i: (i, 0)),
        ],
        out_specs=pl.BlockSpec((RB, C), lambda i: (i, 0)),
        out_shape=jax.ShapeDtypeStruct((B, C), jnp.float32),
    )(lab2, cosine)


# speedup vs baseline: 3.9889x; 3.7567x over previous
import jax
import jax.numpy as jnp
from jax.experimental import pallas as pl

_SCALE = 32.0
_MARGIN = 0.2


def _body(lab_ref, cos_ref, out_ref):
    cos = cos_ref[...]
    lab = lab_ref[...]
    cb = cos.shape[0]
    classes = pl.program_id(0) * cb + jax.lax.broadcasted_iota(
        jnp.int32, cos.shape, 0)
    mask = classes == lab
    out_ref[...] = jnp.where(mask, (cos - _MARGIN) * _SCALE, cos * _SCALE)


def kernel(cosine, label):
    B, C = cosine.shape
    CB = 1000
    cos_t = cosine.T
    lab2 = label.astype(jnp.int32).reshape(1, B)
    out_t = pl.pallas_call(
        _body,
        grid=(C // CB,),
        in_specs=[
            pl.BlockSpec((1, B), lambda i: (0, 0)),
            pl.BlockSpec((CB, B), lambda i: (i, 0)),
        ],
        out_specs=pl.BlockSpec((CB, B), lambda i: (i, 0)),
        out_shape=jax.ShapeDtypeStruct((C, B), jnp.float32),
    )(lab2, cos_t)
    return out_t.T
